# Initial kernel scaffold; baseline (speedup 1.0000x reference)
import jax
import jax.numpy as jnp
from jax.experimental import pallas as pl


def kernel(h_d_prev, h_s, edge_features_embedded, sender_indices, receiver_indices, W1, b1, W2, b2, W):
    raise NotImplementedError("write your pallas kernel here")


# speedup vs baseline: 3.9542x; 3.9542x over previous
import functools

import jax
import jax.numpy as jnp
from jax import lax
from jax.experimental import pallas as pl
from jax.experimental.pallas import tpu as pltpu
from jax.experimental.pallas import tpu_sc as plsc

N = 10000
E = 320000
D = 128
DE = 16
H = 64

NW = 32
EPT = E // NW
C = 80
NWR = 10
RPT = N // NWR
ZC = 40

_mesh = plsc.VectorSubcoreMesh(core_axis_name="c", subcore_axis_name="s")


def _precompute_body(hs_ref, hd_ref, wa_ref, wb_ref, b_ref, pq_ref):
    f32 = jnp.float32
    pq_ref[...] = (jnp.dot(hs_ref[...], wa_ref[...], preferred_element_type=f32)
                   + jnp.dot(hd_ref[...], wb_ref[...], preferred_element_type=f32)
                   + b_ref[...])


def _precompute(h_s, h_d, Wa, Wb, bias):
    NB = 2000
    return pl.pallas_call(
        _precompute_body,
        grid=(N // NB,),
        in_specs=[
            pl.BlockSpec((NB, D), lambda i: (i, 0)),
            pl.BlockSpec((NB, D), lambda i: (i, 0)),
            pl.BlockSpec((D, 2 * H), lambda i: (0, 0)),
            pl.BlockSpec((D, 2 * H), lambda i: (0, 0)),
            pl.BlockSpec((1, 2 * H), lambda i: (0, 0)),
        ],
        out_specs=pl.BlockSpec((NB, 2 * H), lambda i: (i, 0)),
        out_shape=jax.ShapeDtypeStruct((N, 2 * H), jnp.float32),
    )(h_s, h_d, Wa, Wb, bias.reshape(1, 2 * H))


@functools.partial(
    pl.kernel,
    mesh=_mesh,
    out_type=jax.ShapeDtypeStruct((E, H), jnp.float32),
    scratch_types=[
        pltpu.VMEM((C,), jnp.int32),
        pltpu.VMEM((C,), jnp.int32),
        pltpu.VMEM((C, 2 * H), jnp.float32),
        pltpu.VMEM((C, 2 * H), jnp.float32),
        pltpu.VMEM((C, H), jnp.float32),
        pltpu.SemaphoreType.DMA,
        pltpu.SemaphoreType.DMA,
    ],
)
def _gather_combine(pq_hbm, sidx_hbm, ridx_hbm, out_hbm,
                    sidx_v, ridx_v, bufs, bufr, bufo, sem1, sem2):
    wid = lax.axis_index("s") * 2 + lax.axis_index("c")
    base = wid * EPT

    def chunk(j, carry):
        off = pl.multiple_of(base + j * C, 8)
        pltpu.sync_copy(sidx_hbm.at[pl.ds(off, C)], sidx_v)
        pltpu.sync_copy(ridx_hbm.at[pl.ds(off, C)], ridx_v)
        cp1 = pltpu.async_copy(pq_hbm.at[sidx_v], bufs, sem1)
        cp2 = pltpu.async_copy(pq_hbm.at[ridx_v], bufr, sem2)
        cp1.wait()
        cp2.wait()

        def row(r, c2):
            for k in range(H // 16):
                sl = pl.ds(k * 16, 16)
                bufo[r, sl] = bufs[r, pl.ds(k * 16, 16)] + bufr[r, pl.ds(H + k * 16, 16)]
            return c2

        lax.fori_loop(0, C, row, 0)
        pltpu.sync_copy(bufo, out_hbm.at[pl.ds(off, C)])
        return carry

    lax.fori_loop(0, EPT // C, chunk, 0)


def _mlp_body(hpre_ref, ef_ref, w1e_ref, w2_ref, b2_ref, out_ref):
    f32 = jnp.float32
    hidden = jnp.maximum(
        hpre_ref[...] + jnp.dot(ef_ref[...], w1e_ref[...], preferred_element_type=f32),
        0.0)
    out_ref[...] = jnp.maximum(
        jnp.dot(hidden, w2_ref[...], preferred_element_type=f32) + b2_ref[...],
        0.0)


def _edge_mlp(hpre, ef, W1e, W2, b2):
    EB = 2000
    return pl.pallas_call(
        _mlp_body,
        grid=(E // EB,),
        in_specs=[
            pl.BlockSpec((EB, H), lambda i: (i, 0)),
            pl.BlockSpec((EB, DE), lambda i: (i, 0)),
            pl.BlockSpec((DE, H), lambda i: (0, 0)),
            pl.BlockSpec((H, D), lambda i: (0, 0)),
            pl.BlockSpec((1, D), lambda i: (0, 0)),
        ],
        out_specs=pl.BlockSpec((EB, D), lambda i: (i, 0)),
        out_shape=jax.ShapeDtypeStruct((E, D), jnp.float32),
    )(hpre, ef, W1e, W2, b2.reshape(1, D))


@functools.partial(
    pl.kernel,
    mesh=_mesh,
    out_type=jax.ShapeDtypeStruct((2, N, D), jnp.float32),
    scratch_types=[
        pltpu.VMEM((C,), jnp.int32),
        pltpu.VMEM((C,), jnp.int32),
        pltpu.VMEM((C, D), jnp.float32),
        pltpu.VMEM((C, D), jnp.float32),
        pltpu.VMEM((C, D), jnp.float32),
        pltpu.VMEM((ZC, D), jnp.float32),
        pltpu.VMEM_SHARED((N, D), jnp.float32),
        pltpu.SemaphoreType.DMA,
        pltpu.SemaphoreType.DMA,
    ],
)
def _flux_scatter(hd_hbm, psi_hbm, sidx_hbm, ridx_hbm, out_hbm,
                  sidx_v, ridx_v, bufr, bufs, psiv, zbuf, agg, sem1, sem2):
    cid = lax.axis_index("c")
    sid = lax.axis_index("s")
    wid = sid * 2 + cid

    zero = jnp.zeros((16,), jnp.float32)

    def zrow(r, c2):
        for k in range(D // 16):
            zbuf[r, pl.ds(k * 16, 16)] = zero
        return c2

    lax.fori_loop(0, ZC, zrow, 0)

    @pl.when(sid < NWR)
    def _zero_agg():
        def zcopy(t, c2):
            off = pl.multiple_of(sid * RPT + t * ZC, 8)
            pltpu.sync_copy(zbuf, agg.at[pl.ds(off, ZC)])
            return c2

        lax.fori_loop(0, RPT // ZC, zcopy, 0)

    plsc.subcore_barrier()

    base = wid * EPT

    def chunk(j, carry):
        off = pl.multiple_of(base + j * C, 8)
        pltpu.sync_copy(sidx_hbm.at[pl.ds(off, C)], sidx_v)
        pltpu.sync_copy(ridx_hbm.at[pl.ds(off, C)], ridx_v)
        cp1 = pltpu.async_copy(hd_hbm.at[ridx_v], bufr, sem1)
        cp2 = pltpu.async_copy(hd_hbm.at[sidx_v], bufs, sem2)
        pltpu.sync_copy(psi_hbm.at[pl.ds(off, C)], psiv)
        cp1.wait()
        cp2.wait()

        def row(r, c2):
            for k in range(D // 16):
                sl = pl.ds(k * 16, 16)
                bufr[r, sl] = psiv[r, sl] * (bufr[r, sl] - bufs[r, sl])
            return c2

        lax.fori_loop(0, C, row, 0)
        pltpu.sync_copy(bufr, agg.at[ridx_v], add=True)
        return carry

    lax.fori_loop(0, EPT // C, chunk, 0)
    plsc.subcore_barrier()

    @pl.when(sid < NWR)
    def _write_out():
        off0 = pl.multiple_of(sid * RPT, 8)
        sl = pl.ds(off0, RPT)
        pltpu.sync_copy(agg.at[sl], out_hbm.at[cid, sl])


def _final_body(hd_ref, pa_ref, w_ref, out_ref):
    agg = pa_ref[0] + pa_ref[1]
    out_ref[...] = hd_ref[...] + jnp.dot(agg, w_ref[...],
                                         preferred_element_type=jnp.float32)


def _finalize(h_d, partials, W):
    NB = 2000
    return pl.pallas_call(
        _final_body,
        grid=(N // NB,),
        in_specs=[
            pl.BlockSpec((NB, D), lambda i: (i, 0)),
            pl.BlockSpec((2, NB, D), lambda i: (0, i, 0)),
            pl.BlockSpec((D, D), lambda i: (0, 0)),
        ],
        out_specs=pl.BlockSpec((NB, D), lambda i: (i, 0)),
        out_shape=jax.ShapeDtypeStruct((N, D), jnp.float32),
    )(h_d, partials, W)


def kernel(h_d_prev, h_s, edge_features_embedded, sender_indices,
           receiver_indices, W1, b1, W2, b2, W):
    Wa = jnp.concatenate([W1[0:D], W1[D:2 * D]], axis=1)
    Wb = jnp.concatenate([W1[2 * D:3 * D], W1[3 * D:4 * D]], axis=1)
    bias = jnp.concatenate([jnp.zeros_like(b1), b1])
    PQ = _precompute(h_s, h_d_prev, Wa, Wb, bias)
    hpre = _gather_combine(PQ, sender_indices, receiver_indices)
    psi = _edge_mlp(hpre, edge_features_embedded, W1[4 * D:], W2, b2)
    partials = _flux_scatter(h_d_prev, psi, sender_indices, receiver_indices)
    return _finalize(h_d_prev, partials, W)


# speedup vs baseline: 6.3112x; 1.5961x over previous
import functools

import jax
import jax.numpy as jnp
from jax import lax
from jax.experimental import pallas as pl
from jax.experimental.pallas import tpu as pltpu
from jax.experimental.pallas import tpu_sc as plsc

N = 10000
E = 320000
D = 128
DE = 16
H = 64

NW = 32
EPT = E // NW
CB = 80
CD = 40
NWR = 10
RPT = N // NWR
ZC = 8

_mesh = plsc.VectorSubcoreMesh(core_axis_name="c", subcore_axis_name="s")


def _precompute_body(hs_ref, hd_ref, wa_ref, wb_ref, b_ref, pq_ref):
    f32 = jnp.float32
    pq_ref[...] = (jnp.dot(hs_ref[...], wa_ref[...], preferred_element_type=f32)
                   + jnp.dot(hd_ref[...], wb_ref[...], preferred_element_type=f32)
                   + b_ref[...])


def _precompute(h_s, h_d, Wa, Wb, bias):
    NB = 2000
    return pl.pallas_call(
        _precompute_body,
        grid=(N // NB,),
        in_specs=[
            pl.BlockSpec((NB, D), lambda i: (i, 0)),
            pl.BlockSpec((NB, D), lambda i: (i, 0)),
            pl.BlockSpec((D, 2 * H), lambda i: (0, 0)),
            pl.BlockSpec((D, 2 * H), lambda i: (0, 0)),
            pl.BlockSpec((1, 2 * H), lambda i: (0, 0)),
        ],
        out_specs=pl.BlockSpec((NB, 2 * H), lambda i: (i, 0)),
        out_shape=jax.ShapeDtypeStruct((N, 2 * H), jnp.float32),
    )(h_s, h_d, Wa, Wb, bias.reshape(1, 2 * H))


_CHB = EPT // CB


@functools.partial(
    pl.kernel,
    mesh=_mesh,
    out_type=jax.ShapeDtypeStruct((E, H), jnp.float32),
    scratch_types=[
        pltpu.VMEM((CB,), jnp.int32),
        pltpu.VMEM((CB,), jnp.int32),
        pltpu.VMEM((CB,), jnp.int32),
        pltpu.VMEM((CB,), jnp.int32),
        pltpu.VMEM((CB, 2 * H), jnp.float32),
        pltpu.VMEM((CB, 2 * H), jnp.float32),
        pltpu.VMEM((CB, 2 * H), jnp.float32),
        pltpu.VMEM((CB, 2 * H), jnp.float32),
        pltpu.VMEM((CB, H), jnp.float32),
        pltpu.VMEM((CB, H), jnp.float32),
        pltpu.SemaphoreType.DMA,
        pltpu.SemaphoreType.DMA,
        pltpu.SemaphoreType.DMA,
        pltpu.SemaphoreType.DMA,
        pltpu.SemaphoreType.DMA,
        pltpu.SemaphoreType.DMA,
    ],
)
def _gather_combine(pq_hbm, sidx_hbm, ridx_hbm, out_hbm,
                    si0, si1, ri0, ri1, bs0, bs1, br0, br1, res0, res1,
                    semi0, semi1, semg0, semg1, sems0, sems1):
    wid = lax.axis_index("s") * 2 + lax.axis_index("c")
    base = wid * EPT
    sidx = [si0, si1]
    ridx = [ri0, ri1]
    bufs = [bs0, bs1]
    bufr = [br0, br1]
    resv = [res0, res1]
    semi = [semi0, semi1]
    semg = [semg0, semg1]
    sems = [sems0, sems1]

    def eoff(c):
        return pl.multiple_of(base + c * CB, 8)

    def fire_idx(c, b):
        pltpu.async_copy(sidx_hbm.at[pl.ds(eoff(c), CB)], sidx[b], semi[b])
        pltpu.async_copy(ridx_hbm.at[pl.ds(eoff(c), CB)], ridx[b], semi[b])

    def fire_gather(c, b):
        pltpu.make_async_copy(sidx_hbm.at[pl.ds(eoff(c), CB)], sidx[b], semi[b]).wait()
        pltpu.make_async_copy(ridx_hbm.at[pl.ds(eoff(c), CB)], ridx[b], semi[b]).wait()
        pltpu.async_copy(pq_hbm.at[sidx[b]], bufs[b], semg[b])
        pltpu.async_copy(pq_hbm.at[ridx[b]], bufr[b], semg[b])

    def proc_a(c, b):
        pltpu.make_async_copy(pq_hbm.at[sidx[b]], bufs[b], semg[b]).wait()
        pltpu.make_async_copy(pq_hbm.at[ridx[b]], bufr[b], semg[b]).wait()

    def proc_b(c, b):
        @pl.when(c >= 2)
        def _drain_store():
            pltpu.make_async_copy(
                resv[b], out_hbm.at[pl.ds(eoff(c - 2), CB)], sems[b]).wait()

        def row(r, c2):
            for k in range(H // 16):
                resv[b][r, pl.ds(k * 16, 16)] = (
                    bufs[b][r, pl.ds(k * 16, 16)]
                    + bufr[b][r, pl.ds(H + k * 16, 16)])
            return c2

        lax.fori_loop(0, CB, row, 0)
        pltpu.async_copy(resv[b], out_hbm.at[pl.ds(eoff(c), CB)], sems[b])

    fire_idx(0, 0)
    fire_idx(1, 1)
    fire_gather(0, 0)

    def pair(g, carry):
        a = g * 2
        fire_gather(a + 1, 1)
        proc_a(a, 0)

        @pl.when(a + 2 < _CHB)
        def _f0():
            fire_idx(a + 2, 0)

        proc_b(a, 0)

        @pl.when(a + 2 < _CHB)
        def _g0():
            fire_gather(a + 2, 0)

        proc_a(a + 1, 1)

        @pl.when(a + 3 < _CHB)
        def _f1():
            fire_idx(a + 3, 1)

        proc_b(a + 1, 1)
        return carry

    lax.fori_loop(0, _CHB // 2, pair, 0)
    if _CHB % 2 == 1:
        proc_a(_CHB - 1, 0)
        proc_b(_CHB - 1, 0)
    pltpu.make_async_copy(
        resv[0], out_hbm.at[pl.ds(eoff(_CHB - 1), CB)], sems[0]).wait()
    pltpu.make_async_copy(
        resv[1], out_hbm.at[pl.ds(eoff(_CHB - 2), CB)], sems[1]).wait()


def _mlp_body(hpre_ref, ef_ref, w1e_ref, w2_ref, b2_ref, out_ref):
    f32 = jnp.float32
    hidden = jnp.maximum(
        hpre_ref[...] + jnp.dot(ef_ref[...], w1e_ref[...], preferred_element_type=f32),
        0.0)
    out_ref[...] = jnp.maximum(
        jnp.dot(hidden, w2_ref[...], preferred_element_type=f32) + b2_ref[...],
        0.0)


def _edge_mlp(hpre, ef, W1e, W2, b2):
    EB = 2000
    return pl.pallas_call(
        _mlp_body,
        grid=(E // EB,),
        in_specs=[
            pl.BlockSpec((EB, H), lambda i: (i, 0)),
            pl.BlockSpec((EB, DE), lambda i: (i, 0)),
            pl.BlockSpec((DE, H), lambda i: (0, 0)),
            pl.BlockSpec((H, D), lambda i: (0, 0)),
            pl.BlockSpec((1, D), lambda i: (0, 0)),
        ],
        out_specs=pl.BlockSpec((EB, D), lambda i: (i, 0)),
        out_shape=jax.ShapeDtypeStruct((E, D), jnp.float32),
    )(hpre, ef, W1e, W2, b2.reshape(1, D))


_CHD = EPT // CD


@functools.partial(
    pl.kernel,
    mesh=_mesh,
    out_type=jax.ShapeDtypeStruct((2, N, D), jnp.float32),
    scratch_types=[
        pltpu.VMEM((CD,), jnp.int32),
        pltpu.VMEM((CD,), jnp.int32),
        pltpu.VMEM((CD,), jnp.int32),
        pltpu.VMEM((CD,), jnp.int32),
        pltpu.VMEM((CD,), jnp.int32),
        pltpu.VMEM((CD,), jnp.int32),
        pltpu.VMEM((CD, D), jnp.float32),
        pltpu.VMEM((CD, D), jnp.float32),
        pltpu.VMEM((CD, D), jnp.float32),
        pltpu.VMEM((CD, D), jnp.float32),
        pltpu.VMEM((CD, D), jnp.float32),
        pltpu.VMEM((CD, D), jnp.float32),
        pltpu.VMEM((CD, D), jnp.float32),
        pltpu.VMEM((CD, D), jnp.float32),
        pltpu.VMEM((ZC, D), jnp.float32),
        pltpu.VMEM_SHARED((N, D), jnp.float32),
        pltpu.SemaphoreType.DMA,
        pltpu.SemaphoreType.DMA,
        pltpu.SemaphoreType.DMA,
        pltpu.SemaphoreType.DMA,
        pltpu.SemaphoreType.DMA,
        pltpu.SemaphoreType.DMA,
        pltpu.SemaphoreType.DMA,
        pltpu.SemaphoreType.DMA,
    ],
)
def _flux_scatter(hd_hbm, psi_hbm, sidx_hbm, ridx_hbm, out_hbm,
                  si0, si1, ri0, ri1, rs0, rs1, br0, br1, bs0, bs1, ps0, ps1,
                  res0, res1, zbuf, agg,
                  semi0, semi1, semr0, semr1, semg0, semg1, sems0, sems1):
    cid = lax.axis_index("c")
    sid = lax.axis_index("s")
    wid = sid * 2 + cid
    sidx = [si0, si1]
    ridx = [ri0, ri1]
    rsidx = [rs0, rs1]
    bufr = [br0, br1]
    bufs = [bs0, bs1]
    psiv = [ps0, ps1]
    resv = [res0, res1]
    semi = [semi0, semi1]
    semr = [semr0, semr1]
    semg = [semg0, semg1]
    sems = [sems0, sems1]

    zero = jnp.zeros((16,), jnp.float32)

    def zrow(r, c2):
        for k in range(D // 16):
            zbuf[r, pl.ds(k * 16, 16)] = zero
        return c2

    lax.fori_loop(0, ZC, zrow, 0)

    @pl.when(sid < NWR)
    def _zero_agg():
        def zcopy(t, c2):
            off = pl.multiple_of(sid * RPT + t * ZC, 8)
            pltpu.sync_copy(zbuf, agg.at[pl.ds(off, ZC)])
            return c2

        lax.fori_loop(0, RPT // ZC, zcopy, 0)

    plsc.subcore_barrier()

    base = wid * EPT

    def eoff(c):
        return pl.multiple_of(base + c * CD, 8)

    def fire_idx(c, b):
        pltpu.async_copy(sidx_hbm.at[pl.ds(eoff(c), CD)], sidx[b], semi[b])
        pltpu.async_copy(ridx_hbm.at[pl.ds(eoff(c), CD)], ridx[b], semi[b])

    def fire_gather(c, b):
        pltpu.make_async_copy(sidx_hbm.at[pl.ds(eoff(c), CD)], sidx[b], semi[b]).wait()
        pltpu.make_async_copy(ridx_hbm.at[pl.ds(eoff(c), CD)], ridx[b], semi[b]).wait()
        pltpu.async_copy(hd_hbm.at[sidx[b]], bufs[b], semg[b])
        pltpu.async_copy(hd_hbm.at[ridx[b]], bufr[b], semg[b])
        pltpu.async_copy(psi_hbm.at[pl.ds(eoff(c), CD)], psiv[b], semg[b])

    def proc_a(c, b):
        @pl.when(c >= 2)
        def _drain_scatter():
            pltpu.make_async_copy(resv[b], agg.at[rsidx[b]], sems[b]).wait()

        pltpu.make_async_copy(hd_hbm.at[sidx[b]], bufs[b], semg[b]).wait()
        pltpu.make_async_copy(hd_hbm.at[ridx[b]], bufr[b], semg[b]).wait()
        pltpu.make_async_copy(psi_hbm.at[pl.ds(eoff(c), CD)], psiv[b], semg[b]).wait()
        pltpu.async_copy(ridx_hbm.at[pl.ds(eoff(c), CD)], rsidx[b], semr[b])

    def proc_b(c, b):
        def row(r, c2):
            for k in range(D // 16):
                sl = pl.ds(k * 16, 16)
                resv[b][r, sl] = psiv[b][r, sl] * (bufr[b][r, sl] - bufs[b][r, sl])
            return c2

        lax.fori_loop(0, CD, row, 0)
        pltpu.make_async_copy(ridx_hbm.at[pl.ds(eoff(c), CD)], rsidx[b], semr[b]).wait()
        pltpu.async_copy(resv[b], agg.at[rsidx[b]], sems[b], add=True)

    fire_idx(0, 0)
    fire_idx(1, 1)
    fire_gather(0, 0)

    def pair(g, carry):
        a = g * 2
        fire_gather(a + 1, 1)
        proc_a(a, 0)

        @pl.when(a + 2 < _CHD)
        def _f0():
            fire_idx(a + 2, 0)

        proc_b(a, 0)

        @pl.when(a + 2 < _CHD)
        def _g0():
            fire_gather(a + 2, 0)

        proc_a(a + 1, 1)

        @pl.when(a + 3 < _CHD)
        def _f1():
            fire_idx(a + 3, 1)

        proc_b(a + 1, 1)
        return carry

    lax.fori_loop(0, _CHD // 2, pair, 0)
    if _CHD % 2 == 1:
        proc_a(_CHD - 1, 0)
        proc_b(_CHD - 1, 0)
    pltpu.make_async_copy(resv[0], agg.at[rsidx[0]], sems[0]).wait()
    pltpu.make_async_copy(resv[1], agg.at[rsidx[1]], sems[1]).wait()
    plsc.subcore_barrier()

    @pl.when(sid < NWR)
    def _write_out():
        off0 = pl.multiple_of(sid * RPT, 8)
        sl = pl.ds(off0, RPT)
        pltpu.sync_copy(agg.at[sl], out_hbm.at[cid, sl])


def _final_body(hd_ref, pa_ref, w_ref, out_ref):
    agg = pa_ref[0] + pa_ref[1]
    out_ref[...] = hd_ref[...] + jnp.dot(agg, w_ref[...],
                                         preferred_element_type=jnp.float32)


def _finalize(h_d, partials, W):
    NB = 2000
    return pl.pallas_call(
        _final_body,
        grid=(N // NB,),
        in_specs=[
            pl.BlockSpec((NB, D), lambda i: (i, 0)),
            pl.BlockSpec((2, NB, D), lambda i: (0, i, 0)),
            pl.BlockSpec((D, D), lambda i: (0, 0)),
        ],
        out_specs=pl.BlockSpec((NB, D), lambda i: (i, 0)),
        out_shape=jax.ShapeDtypeStruct((N, D), jnp.float32),
    )(h_d, partials, W)


def kernel(h_d_prev, h_s, edge_features_embedded, sender_indices,
           receiver_indices, W1, b1, W2, b2, W):
    Wa = jnp.concatenate([W1[0:D], W1[D:2 * D]], axis=1)
    Wb = jnp.concatenate([W1[2 * D:3 * D], W1[3 * D:4 * D]], axis=1)
    bias = jnp.concatenate([jnp.zeros_like(b1), b1])
    PQ = _precompute(h_s, h_d_prev, Wa, Wb, bias)
    hpre = _gather_combine(PQ, sender_indices, receiver_indices)
    psi = _edge_mlp(hpre, edge_features_embedded, W1[4 * D:], W2, b2)
    partials = _flux_scatter(h_d_prev, psi, sender_indices, receiver_indices)
    return _finalize(h_d_prev, partials, W)
